# Initial kernel scaffold; baseline (speedup 1.0000x reference)
import jax
import jax.numpy as jnp
from jax.experimental import pallas as pl


def kernel(ref_corr_indices, src_corr_indices, ref_no_match_indices, src_no_match_indices, gt_corr, gt_corr_score, voxels):
    raise NotImplementedError("write your pallas kernel here")


# speedup vs baseline: 1.8233x; 1.8233x over previous
import functools
import math

import jax
import jax.numpy as jnp
from jax import lax
from jax.experimental import pallas as pl
from jax.experimental.pallas import tpu as pltpu
from jax.experimental.pallas import tpu_sc as plsc

R = 4096
S = 4096
DIM = 128
N_CORR = 16384
N_NOMATCH = 1024
N_GT = 16384
N_VOX = 4096
RS = R * S

_NC = 2
_NS = 16

_CORR_PER_TILE = N_CORR // _NS
_NM_PER_TILE = N_NOMATCH // _NS
_GT_PER_TILE = N_GT // _NS
_ZCHUNK = 32768
_ZPER_TILE = RS // _NS // _ZCHUNK
_ROWS = _CORR_PER_TILE // 128


def _sc_body(rci, sci, rnm, snm, gtk, gtv, m1, m2,
             zbuf, ia, ib, fidx, kf, vbuf, ones128, nmi, nmf, nmv, sem):
    c = lax.axis_index("c")
    s = lax.axis_index("s")

    zeros16 = jnp.zeros((16,), jnp.float32)
    ones16 = jnp.ones((16,), jnp.float32)

    def _fill_z(i, _):
        zbuf[pl.ds(i * 16, 16)] = zeros16
        return 0

    lax.fori_loop(0, _ZCHUNK // 16, _fill_z, 0)
    for j in range(8):
        ones128[pl.ds(j * 16, 16)] = ones16
    for j in range(4):
        nmv[pl.ds(j * 16, 16)] = ones16

    def _zero(out):
        base = s * (RS // _NS)
        for wave in range(_ZPER_TILE // 8):
            ds = []
            for b in range(8):
                w = wave * 8 + b
                ds.append(pltpu.async_copy(
                    zbuf, out.at[pl.ds(base + w * _ZCHUNK, _ZCHUNK)], sem))
            for d in ds:
                d.wait()

    @pl.when(c == 0)
    def _():
        _zero(m1)

    @pl.when(c == 1)
    def _():
        _zero(m2)

    plsc.subcore_barrier()

    @pl.when(c == 0)
    def _():
        base = s * _CORR_PER_TILE
        pltpu.sync_copy(rci.at[pl.ds(base, _CORR_PER_TILE)], ia)
        pltpu.sync_copy(sci.at[pl.ds(base, _CORR_PER_TILE)], ib)
        for r in range(_ROWS):
            for j in range(8):
                o = r * 128 + j * 16
                fidx[r, pl.ds(j * 16, 16)] = ia[pl.ds(o, 16)] * S + ib[pl.ds(o, 16)]
        ds = [pltpu.async_copy(ones128, m1.at[fidx.at[r]], sem)
              for r in range(_ROWS)]
        for d in ds:
            d.wait()
        nbase = s * _NM_PER_TILE
        pltpu.sync_copy(rnm.at[pl.ds(nbase, _NM_PER_TILE)], nmi)
        for j in range(_NM_PER_TILE // 16):
            nmf[pl.ds(j * 16, 16)] = nmi[pl.ds(j * 16, 16)] * S
        pltpu.sync_copy(snm.at[pl.ds(nbase, _NM_PER_TILE)], nmi)
        for j in range(_NM_PER_TILE // 16):
            nmf[pl.ds(j * 16, 16)] = nmf[pl.ds(j * 16, 16)] + nmi[pl.ds(j * 16, 16)]
        pltpu.async_copy(nmv, m1.at[nmf], sem).wait()

    @pl.when(c == 1)
    def _():
        base = s * _GT_PER_TILE
        ds = []
        for r in range(_ROWS):
            ds.append(pltpu.async_copy(
                gtk.at[pl.ds(base + r * 128, 128)], kf.at[r], sem))
            ds.append(pltpu.async_copy(
                gtv.at[pl.ds(base + r * 128, 128)], vbuf.at[r], sem))
        for d in ds:
            d.wait()
        ds = [pltpu.async_copy(vbuf.at[r], m2.at[kf.at[r]], sem)
              for r in range(_ROWS)]
        for d in ds:
            d.wait()


_sc_build = functools.partial(
    pl.kernel,
    out_type=(
        jax.ShapeDtypeStruct((RS,), jnp.float32),
        jax.ShapeDtypeStruct((RS,), jnp.float32),
    ),
    mesh=plsc.VectorSubcoreMesh(
        core_axis_name="c", subcore_axis_name="s",
        num_cores=_NC, num_subcores=_NS),
    scratch_types=[
        pltpu.VMEM((_ZCHUNK,), jnp.float32),
        pltpu.VMEM((_CORR_PER_TILE,), jnp.int32),
        pltpu.VMEM((_CORR_PER_TILE,), jnp.int32),
        pltpu.VMEM((_ROWS, 128), jnp.int32),
        pltpu.VMEM((_ROWS, 128), jnp.int32),
        pltpu.VMEM((_ROWS, 128), jnp.float32),
        pltpu.VMEM((128,), jnp.float32),
        pltpu.VMEM((_NM_PER_TILE,), jnp.int32),
        pltpu.VMEM((_NM_PER_TILE,), jnp.int32),
        pltpu.VMEM((_NM_PER_TILE,), jnp.float32),
        pltpu.SemaphoreType.DMA,
    ],
)(_sc_body)


def _emb_body(vox_ref, out_ref):
    part = DIM // 6
    scale = math.log(10000.0) / (part - 1)
    x = vox_ref[:, 0:1]
    y = vox_ref[:, 1:2]
    z = vox_ref[:, 2:3]
    d = lax.broadcasted_iota(jnp.int32, (N_VOX, DIM), 1)
    g = d // part
    j = d - g * part
    freq = jnp.exp(j.astype(jnp.float32) * (-scale))
    coord = jnp.where(g < 2, x, jnp.where(g < 4, y, z))
    t = coord * freq
    val = jnp.where(g % 2 == 0, jnp.sin(t), jnp.cos(t))
    out_ref[...] = jnp.where(d < 6 * part, val, 0.0)


_emb_build = pl.pallas_call(
    _emb_body,
    out_shape=jax.ShapeDtypeStruct((N_VOX, DIM), jnp.float32),
)


def kernel(ref_corr_indices, src_corr_indices, ref_no_match_indices,
           src_no_match_indices, gt_corr, gt_corr_score, voxels):
    rci = ref_corr_indices.astype(jnp.int32)
    sci = src_corr_indices.astype(jnp.int32)
    rnm = ref_no_match_indices.astype(jnp.int32)
    snm = src_no_match_indices.astype(jnp.int32)
    rows = gt_corr[:, 0].astype(jnp.int32)
    cols = gt_corr[:, 1].astype(jnp.int32)
    score = gt_corr_score.astype(jnp.float32)

    flat = rows * S + cols
    ks, vs = lax.sort((flat, score), num_keys=1, is_stable=False)
    n = N_GT
    pos = jnp.arange(n, dtype=jnp.int32)
    is_last = jnp.concatenate([ks[1:] != ks[:-1], jnp.array([True])])
    lastpos = jnp.flip(lax.cummin(jnp.flip(jnp.where(is_last, pos, n - 1))))
    wv = vs[lastpos]

    m1, m2 = _sc_build(rci, sci, rnm, snm, ks, wv)
    emb = _emb_build(voxels.reshape(N_VOX, 3).astype(jnp.float32))
    return (m1.reshape(R, S), m2.reshape(R, S), emb.reshape(1, N_VOX, DIM))
